# baseline (speedup 1.0000x reference)
import functools

import jax
import jax.numpy as jnp
from jax import lax
from jax.experimental import pallas as pl
from jax.experimental.pallas import tpu as pltpu
from jax.experimental.pallas import tpu_sc as plsc

HIDDEN = 64
NC, NS = 2, 16
NW = NC * NS
GSZ = 128


def _sc_gather_add_type(table, type_table, idx2d, typ2d, rows):
    ng_total = idx2d.shape[0]
    ng = ng_total // NW
    NBUF = 4
    LOOK = 2
    lines = rows // 2
    lpc = GSZ // 2
    lpw = ng * lpc

    mesh = plsc.VectorSubcoreMesh(core_axis_name="c", subcore_axis_name="s")

    @functools.partial(
        pl.kernel,
        mesh=mesh,
        compiler_params=pltpu.CompilerParams(
            use_tc_tiling_on_sc=False, needs_layout_passes=False
        ),
        out_type=jax.ShapeDtypeStruct((lines, 2 * HIDDEN), jnp.float32),
        scratch_types=[
            pltpu.VMEM((ng, GSZ), jnp.int32),
            pltpu.VMEM((ng, GSZ), jnp.int32),
            pltpu.VMEM((NBUF, GSZ, HIDDEN), jnp.float32),
            pltpu.VMEM((NBUF, lpc, 2 * HIDDEN), jnp.float32),
            pltpu.VMEM((4, HIDDEN), jnp.float32),
            pltpu.SemaphoreType.DMA((NBUF,)),
            pltpu.SemaphoreType.DMA((NBUF,)),
        ],
    )
    def k(table_hbm, ttab_hbm, idx_hbm, typ_hbm, out_hbm,
          idx_v, typ_v, g_v, o_v, t4_v, gsem, osem):
        wid = lax.axis_index("s") * NC + lax.axis_index("c")
        pltpu.sync_copy(idx_hbm.at[pl.ds(wid * ng, ng)], idx_v)
        pltpu.sync_copy(typ_hbm.at[pl.ds(wid * ng, ng)], typ_v)
        pltpu.sync_copy(ttab_hbm, t4_v)

        iota16 = lax.iota(jnp.int32, 16)
        lanebase = (iota16 & 1) * HIDDEN
        rowv = [iota16 + 16 * q for q in range(8)]
        linev = [(iota16 >> 1) + 8 * q for q in range(8)]

        def fire_gather(j, b):
            pltpu.async_copy(table_hbm.at[idx_v.at[j]], g_v.at[b], gsem.at[b])

        def wait_gather(b):
            pltpu.make_async_copy(
                table_hbm.at[pl.ds(0, GSZ)], g_v.at[b], gsem.at[b]
            ).wait()

        def wait_outcopy(b):
            pltpu.make_async_copy(
                o_v.at[b], out_hbm.at[pl.ds(0, lpc)], osem.at[b]
            ).wait()

        for j0 in range(LOOK):
            fire_gather(j0, j0)

        def body(j, carry):
            b = lax.rem(j, NBUF)
            wait_gather(b)
            jn = j + LOOK
            bn = lax.rem(jn, NBUF)

            @pl.when(jn < ng)
            def _():
                @pl.when(j >= NBUF - LOOK)
                def _():
                    wait_outcopy(bn)

                fire_gather(jn, bn)

            tv = [typ_v[j, pl.ds(16 * q, 16)] for q in range(8)]

            def col(c, cc):
                cvec = jnp.zeros((16,), jnp.int32) + c
                lanev = lanebase + c
                for q in range(8):
                    tt = plsc.load_gather(t4_v, [tv[q], cvec])
                    x = plsc.load_gather(g_v.at[b], [rowv[q], cvec])
                    plsc.store_scatter(o_v.at[b], [linev[q], lanev], x + tt)
                return cc

            lax.fori_loop(0, HIDDEN, col, 0)

            pltpu.async_copy(
                o_v.at[b],
                out_hbm.at[pl.ds(wid * lpw + j * lpc, lpc)],
                osem.at[b],
            )
            return carry

        lax.fori_loop(0, ng, body, 0)
        for b in range(NBUF):
            wait_outcopy(b)

    return k(table, type_table, idx2d, typ2d)


def _tc_ln_pair(emb2, pos2, gamma2, beta2, B, S):
    lines = emb2.shape[0]
    R2 = 3200
    BB = 2 * R2 // S

    def body(x_ref, p_ref, g_ref, b_ref, o_ref):
        x = x_ref[...] + p_ref[...]
        i0 = lax.broadcasted_iota(jnp.int32, (128, 128), 0)
        i1 = lax.broadcasted_iota(jnp.int32, (128, 128), 1)
        m = jnp.where((i0 // HIDDEN) == (i1 // HIDDEN), 1.0 / HIDDEN, 0.0)
        mean = jnp.dot(x, m, preferred_element_type=jnp.float32)
        msq = jnp.dot(x * x, m, preferred_element_type=jnp.float32)
        var = msq - mean * mean
        y = (x - mean) * lax.rsqrt(var + 1e-5) * g_ref[...] + b_ref[...]
        o_ref[...] = y

    out2 = pl.pallas_call(
        body,
        grid=(lines // R2,),
        in_specs=[
            pl.BlockSpec((R2, 2 * HIDDEN), lambda i: (i, 0)),
            pl.BlockSpec((R2, 2 * HIDDEN), lambda i: (0, 0)),
            pl.BlockSpec((1, 2 * HIDDEN), lambda i: (0, 0)),
            pl.BlockSpec((1, 2 * HIDDEN), lambda i: (0, 0)),
        ],
        out_specs=pl.BlockSpec((R2, 2 * HIDDEN), lambda i: (i, 0)),
        out_shape=jax.ShapeDtypeStruct((lines, 2 * HIDDEN), jnp.float32),
    )(emb2, pos2, gamma2, beta2)
    return out2.reshape(B, S, HIDDEN)


def kernel(token_ids, type_ids, token_table, type_table, pos_table, ln_gamma, ln_beta):
    B, S = token_ids.shape
    rows = B * S
    idx2d = token_ids.reshape(rows // GSZ, GSZ).astype(jnp.int32)
    typ2d = type_ids.reshape(rows // GSZ, GSZ).astype(jnp.int32)
    emb2 = _sc_gather_add_type(token_table, type_table, idx2d, typ2d, rows)
    pos_pair = pos_table[:S].reshape(S // 2, 2 * HIDDEN)
    pos2 = jnp.tile(pos_pair, (3200 // (S // 2), 1))
    return _tc_ln_pair(
        emb2,
        pos2,
        jnp.concatenate([ln_gamma, ln_gamma]).reshape(1, 2 * HIDDEN),
        jnp.concatenate([ln_beta, ln_beta]).reshape(1, 2 * HIDDEN),
        B,
        S,
    )


# speedup vs baseline: 2.0408x; 2.0408x over previous
import functools

import jax
import jax.numpy as jnp
from jax import lax
from jax.experimental import pallas as pl
from jax.experimental.pallas import tpu as pltpu
from jax.experimental.pallas import tpu_sc as plsc

HIDDEN = 64
NC, NS = 2, 16
NW = NC * NS
GSZ = 128


def _sc_gather_sum(table, comb, idx2d, c2d, rows):
    ng_total = idx2d.shape[0]
    ng = ng_total // NW
    NBUF = 4
    LOOK = 2
    lines = rows // 2
    lpc = GSZ // 2
    lpw = ng * lpc

    mesh = plsc.VectorSubcoreMesh(core_axis_name="c", subcore_axis_name="s")

    @functools.partial(
        pl.kernel,
        mesh=mesh,
        compiler_params=pltpu.CompilerParams(
            use_tc_tiling_on_sc=False, needs_layout_passes=False
        ),
        out_type=jax.ShapeDtypeStruct((lines, 2 * HIDDEN), jnp.float32),
        scratch_types=[
            pltpu.VMEM((ng, GSZ), jnp.int32),
            pltpu.VMEM((ng, lpc), jnp.int32),
            pltpu.VMEM((NBUF, GSZ, HIDDEN), jnp.float32),
            pltpu.VMEM((NBUF, lpc, 2 * HIDDEN), jnp.float32),
            pltpu.SemaphoreType.DMA((NBUF,)),
            pltpu.SemaphoreType.DMA((NBUF,)),
            pltpu.SemaphoreType.DMA((NBUF,)),
        ],
    )
    def k(table_hbm, comb_hbm, idx_hbm, c2_hbm, out_hbm,
          idx_v, c2_v, g_v, c_v, gsem, csem, osem):
        wid = lax.axis_index("s") * NC + lax.axis_index("c")
        pltpu.sync_copy(idx_hbm.at[pl.ds(wid * ng, ng)], idx_v)
        pltpu.sync_copy(c2_hbm.at[pl.ds(wid * ng, ng)], c2_v)

        def fire(j, b):
            pltpu.async_copy(table_hbm.at[idx_v.at[j]], g_v.at[b], gsem.at[b])
            pltpu.async_copy(comb_hbm.at[c2_v.at[j]], c_v.at[b], csem.at[b])

        def wait_gathers(b):
            pltpu.make_async_copy(
                table_hbm.at[pl.ds(0, GSZ)], g_v.at[b], gsem.at[b]
            ).wait()
            pltpu.make_async_copy(
                comb_hbm.at[pl.ds(0, lpc)], c_v.at[b], csem.at[b]
            ).wait()

        def wait_outcopy(b):
            pltpu.make_async_copy(
                c_v.at[b], out_hbm.at[pl.ds(0, lpc)], osem.at[b]
            ).wait()

        for j0 in range(LOOK):
            fire(j0, j0)

        def body(j, carry):
            b = lax.rem(j, NBUF)
            wait_gathers(b)
            jn = j + LOOK
            bn = lax.rem(jn, NBUF)

            @pl.when(jn < ng)
            def _():
                @pl.when(j >= NBUF - LOOK)
                def _():
                    wait_outcopy(bn)

                fire(jn, bn)

            def line(l, cc):
                for q in range(8):
                    r = 2 * l + q // 4
                    sl = pl.ds(16 * (q % 4), 16)
                    dl = pl.ds(16 * q, 16)
                    c_v[b, l, dl] = c_v[b, l, dl] + g_v[b, r, sl]
                return cc

            lax.fori_loop(0, lpc, line, 0)

            pltpu.async_copy(
                c_v.at[b],
                out_hbm.at[pl.ds(wid * lpw + j * lpc, lpc)],
                osem.at[b],
            )
            return carry

        lax.fori_loop(0, ng, body, 0)
        for b in range(NBUF):
            wait_outcopy(b)

    return k(table, comb, idx2d, c2d)


def _tc_ln_pair(emb2, gamma2, beta2):
    lines = emb2.shape[0]
    R2 = 3200

    def body(x_ref, g_ref, b_ref, o_ref):
        x = x_ref[...]
        i0 = lax.broadcasted_iota(jnp.int32, (128, 128), 0)
        i1 = lax.broadcasted_iota(jnp.int32, (128, 128), 1)
        m = jnp.where((i0 // HIDDEN) == (i1 // HIDDEN), 1.0 / HIDDEN, 0.0)
        mean = jnp.dot(x, m, preferred_element_type=jnp.float32)
        msq = jnp.dot(x * x, m, preferred_element_type=jnp.float32)
        var = msq - mean * mean
        o_ref[...] = (x - mean) * lax.rsqrt(var + 1e-5) * g_ref[...] + b_ref[...]

    return pl.pallas_call(
        body,
        grid=(lines // R2,),
        in_specs=[
            pl.BlockSpec((R2, 2 * HIDDEN), lambda i: (i, 0)),
            pl.BlockSpec((1, 2 * HIDDEN), lambda i: (0, 0)),
            pl.BlockSpec((1, 2 * HIDDEN), lambda i: (0, 0)),
        ],
        out_specs=pl.BlockSpec((R2, 2 * HIDDEN), lambda i: (i, 0)),
        out_shape=jax.ShapeDtypeStruct((lines, 2 * HIDDEN), jnp.float32),
    )(emb2, gamma2, beta2)


def kernel(token_ids, type_ids, token_table, type_table, pos_table, ln_gamma, ln_beta):
    B, S = token_ids.shape
    rows = B * S
    lines = rows // 2
    hs = S // 2
    idx2d = token_ids.reshape(rows // GSZ, GSZ).astype(jnp.int32)

    ntypes = type_table.shape[0]
    pos_pair = pos_table[:S].reshape(1, hs, 2 * HIDDEN)
    ta = jnp.repeat(type_table, ntypes, axis=0)
    tb = jnp.tile(type_table, (ntypes, 1))
    tcat = jnp.concatenate([ta, tb], axis=1)
    comb = (tcat[:, None, :] + pos_pair).reshape(ntypes * ntypes * hs, 2 * HIDDEN)

    t2 = type_ids.astype(jnp.int32).reshape(lines, 2)
    pcode = t2[:, 0] * ntypes + t2[:, 1]
    ppos = jax.lax.broadcasted_iota(jnp.int32, (lines,), 0) % hs
    c2d = (pcode * hs + ppos).reshape(rows // GSZ, GSZ // 2)

    emb2 = _sc_gather_sum(token_table, comb, idx2d, c2d, rows)
    out2 = _tc_ln_pair(
        emb2,
        jnp.concatenate([ln_gamma, ln_gamma]).reshape(1, 2 * HIDDEN),
        jnp.concatenate([ln_beta, ln_beta]).reshape(1, 2 * HIDDEN),
    )
    return out2.reshape(B, S, HIDDEN)


# speedup vs baseline: 2.4784x; 1.2145x over previous
import functools

import jax
import jax.numpy as jnp
from jax import lax
from jax.experimental import pallas as pl
from jax.experimental.pallas import tpu as pltpu
from jax.experimental.pallas import tpu_sc as plsc

HIDDEN = 64
NC, NS = 2, 16
NW = NC * NS
GSZ = 128


def _sc_gather_sum(table, comb, idx2d, c2d, rows):
    ng_total = idx2d.shape[0]
    ng = ng_total // NW
    NBUF = 4
    LOOK = 2
    lpc = GSZ // 2
    rpw = ng * GSZ

    mesh = plsc.VectorSubcoreMesh(core_axis_name="c", subcore_axis_name="s")

    @functools.partial(
        pl.kernel,
        mesh=mesh,
        compiler_params=pltpu.CompilerParams(
            use_tc_tiling_on_sc=False, needs_layout_passes=False
        ),
        out_type=jax.ShapeDtypeStruct((rows, 2 * HIDDEN), jnp.float32),
        scratch_types=[
            pltpu.VMEM((ng, GSZ), jnp.int32),
            pltpu.VMEM((ng, lpc), jnp.int32),
            pltpu.VMEM((NBUF, GSZ, HIDDEN), jnp.float32),
            pltpu.VMEM((NBUF, lpc, 2 * HIDDEN), jnp.float32),
            pltpu.SemaphoreType.DMA((NBUF,)),
            pltpu.SemaphoreType.DMA((NBUF,)),
            pltpu.SemaphoreType.DMA((NBUF,)),
        ],
    )
    def k(table_hbm, comb_hbm, idx_hbm, c2_hbm, out_hbm,
          idx_v, c2_v, g_v, c_v, gsem, csem, osem):
        wid = lax.axis_index("s") * NC + lax.axis_index("c")
        pltpu.sync_copy(idx_hbm.at[pl.ds(wid * ng, ng)], idx_v)
        pltpu.sync_copy(c2_hbm.at[pl.ds(wid * ng, ng)], c2_v)

        def fire(j, b):
            pltpu.async_copy(table_hbm.at[idx_v.at[j]], g_v.at[b], gsem.at[b])
            pltpu.async_copy(comb_hbm.at[c2_v.at[j]], c_v.at[b], csem.at[b])

        def wait_gathers(b):
            pltpu.make_async_copy(
                table_hbm.at[pl.ds(0, GSZ)], g_v.at[b], gsem.at[b]
            ).wait()
            pltpu.make_async_copy(
                comb_hbm.at[pl.ds(0, lpc)], c_v.at[b], csem.at[b]
            ).wait()

        def wait_outcopy(b):
            pltpu.make_async_copy(
                g_v.at[b],
                out_hbm.at[pl.ds(0, GSZ), pl.ds(0, HIDDEN)],
                osem.at[b],
            ).wait()

        for j0 in range(LOOK):
            fire(j0, j0)

        def body(j, carry):
            b = lax.rem(j, NBUF)
            wait_gathers(b)
            jn = j + LOOK
            bn = lax.rem(jn, NBUF)

            @pl.when(jn < ng)
            def _():
                @pl.when(j >= NBUF - LOOK)
                def _():
                    wait_outcopy(bn)

                fire(jn, bn)

            for i in range(GSZ):
                for q in range(4):
                    sl = pl.ds(16 * q, 16)
                    cl = pl.ds((i % 2) * HIDDEN + 16 * q, 16)
                    g_v[b, i, sl] = g_v[b, i, sl] + c_v[b, i // 2, cl]

            pltpu.async_copy(
                g_v.at[b],
                out_hbm.at[pl.ds(wid * rpw + j * GSZ, GSZ), pl.ds(0, HIDDEN)],
                osem.at[b],
            )
            return carry

        lax.fori_loop(0, ng, body, 0)
        for b in range(NBUF):
            wait_outcopy(b)

    return k(table, comb, idx2d, c2d)


def _tc_ln(embp, gamma, beta, rows):
    R = 6400

    def body(x_ref, g_ref, b_ref, o_ref):
        xr = x_ref[...]
        lane = lax.broadcasted_iota(jnp.int32, (R, 2 * HIDDEN), 1)
        x = jnp.where(lane < HIDDEN, xr, 0.0)
        i0 = lax.broadcasted_iota(jnp.int32, (2 * HIDDEN, 2 * HIDDEN), 0)
        i1 = lax.broadcasted_iota(jnp.int32, (2 * HIDDEN, 2 * HIDDEN), 1)
        m = jnp.where((i0 // HIDDEN) == (i1 // HIDDEN), 1.0 / HIDDEN, 0.0)
        mean = jnp.dot(x, m, preferred_element_type=jnp.float32)
        msq = jnp.dot(x * x, m, preferred_element_type=jnp.float32)
        var = msq - mean * mean
        y = (x - mean) * lax.rsqrt(var + 1e-5) * g_ref[...] + b_ref[...]
        o_ref[...] = y[:, :HIDDEN]

    return pl.pallas_call(
        body,
        grid=(rows // R,),
        in_specs=[
            pl.BlockSpec((R, 2 * HIDDEN), lambda i: (i, 0)),
            pl.BlockSpec((1, 2 * HIDDEN), lambda i: (0, 0)),
            pl.BlockSpec((1, 2 * HIDDEN), lambda i: (0, 0)),
        ],
        out_specs=pl.BlockSpec((R, HIDDEN), lambda i: (i, 0)),
        out_shape=jax.ShapeDtypeStruct((rows, HIDDEN), jnp.float32),
    )(embp, jnp.tile(gamma, 2).reshape(1, -1), jnp.tile(beta, 2).reshape(1, -1))


def kernel(token_ids, type_ids, token_table, type_table, pos_table, ln_gamma, ln_beta):
    B, S = token_ids.shape
    rows = B * S
    lines = rows // 2
    hs = S // 2
    idx2d = token_ids.reshape(rows // GSZ, GSZ).astype(jnp.int32)

    ntypes = type_table.shape[0]
    pos_pair = pos_table[:S].reshape(1, hs, 2 * HIDDEN)
    ta = jnp.repeat(type_table, ntypes, axis=0)
    tb = jnp.tile(type_table, (ntypes, 1))
    tcat = jnp.concatenate([ta, tb], axis=1)
    comb = (tcat[:, None, :] + pos_pair).reshape(ntypes * ntypes * hs, 2 * HIDDEN)

    t2 = type_ids.astype(jnp.int32).reshape(lines, 2)
    pcode = t2[:, 0] * ntypes + t2[:, 1]
    ppos = jax.lax.broadcasted_iota(jnp.int32, (lines,), 0) % hs
    c2d = (pcode * hs + ppos).reshape(rows // GSZ, GSZ // 2)

    embp = _sc_gather_sum(token_table, comb, idx2d, c2d, rows)
    out = _tc_ln(embp, ln_gamma, ln_beta, rows)
    return out.reshape(B, S, HIDDEN)
